# baseline (speedup 1.0000x reference)
import functools

import jax
import jax.numpy as jnp
from jax import lax
from jax.experimental import pallas as pl
from jax.experimental.pallas import tpu as pltpu
from jax.experimental.pallas import tpu_sc as plsc

B = 16384
D = 128
C = 1000
NC = 2
NS = 16
NW = NC * NS
CHUNK = 128
CP = 1024
CROWS = CP // NS

B_SC = 8192
B_TC = B - B_SC
ROWS_SC = B_SC // NW
NCHUNK_SC = ROWS_SC // CHUNK
LROWS = B // CHUNK // NW
K_TC = 2048


def _sc_segment_sum(features, labels2d):
    mesh = plsc.VectorSubcoreMesh(core_axis_name="c", subcore_axis_name="s")

    @functools.partial(
        pl.kernel,
        mesh=mesh,
        compiler_params=pltpu.CompilerParams(needs_layout_passes=False),
        out_type=(
            jax.ShapeDtypeStruct((NC, CP, D), jnp.float32),
            jax.ShapeDtypeStruct((NW, CP), jnp.float32),
        ),
        scratch_types=[
            pltpu.VMEM((ROWS_SC, D), jnp.float32),
            pltpu.VMEM((NCHUNK_SC, CHUNK), jnp.int32),
            pltpu.VMEM((LROWS, CHUNK), jnp.int32),
            pltpu.VMEM((CP,), jnp.float32),
            pltpu.VMEM_SHARED((CP, D), jnp.float32),
            pltpu.SemaphoreType.DMA,
            [pltpu.SemaphoreType.DMA] * NCHUNK_SC,
            pltpu.SemaphoreType.DMA,
        ],
    )
    def k(feat_hbm, lbl_hbm, zero_hbm, sum_out, cnt_out,
          feat_v, lbls_v, lblc_v, cnt_v, acc_s, sem_l, sems_f, sem_s):
        cid = lax.axis_index("c")
        sid = lax.axis_index("s")
        wid = cid * NS + sid
        base = wid * ROWS_SC

        lcop = pltpu.async_copy(
            lbl_hbm.at[pl.ds(wid * LROWS, LROWS)], lblc_v, sem_l)
        fcops = [
            pltpu.async_copy(
                feat_hbm.at[pl.ds(base + j * CHUNK, CHUNK)],
                feat_v.at[pl.ds(j * CHUNK, CHUNK)], sems_f[j])
            for j in range(NCHUNK_SC)
        ]
        scop = pltpu.async_copy(
            lbl_hbm.at[pl.ds(wid * NCHUNK_SC, NCHUNK_SC)], lbls_v, sem_l)

        zeros16 = jnp.zeros((16,), jnp.float32)
        ones16 = jnp.ones((16,), jnp.float32)

        def fill_zc(i, _):
            cnt_v[pl.ds(i * 16, 16)] = zeros16
            return 0

        lax.fori_loop(0, CP // 16, fill_zc, 0)

        pltpu.sync_copy(zero_hbm.at[pl.ds(sid * CROWS, CROWS)],
                        acc_s.at[pl.ds(sid * CROWS, CROWS)])

        lcop.wait()

        def count_step(v, _):
            lbl16 = lblc_v[v // 8, pl.ds((v % 8) * 16, 16)]
            plsc.addupdate_scatter(cnt_v, [lbl16], ones16)
            return 0

        lax.fori_loop(0, LROWS * 8, count_step, 0)

        scop.wait()
        plsc.subcore_barrier()

        scats = []
        for j in range(NCHUNK_SC):
            fcops[j].wait()
            scats.append(pltpu.async_copy(
                feat_v.at[pl.ds(j * CHUNK, CHUNK)],
                acc_s.at[lbls_v.at[j]], sem_s, add=True))
        for s in scats:
            s.wait()

        plsc.subcore_barrier()

        r0 = sid * CROWS
        pltpu.sync_copy(acc_s.at[pl.ds(r0, CROWS)],
                        sum_out.at[cid, pl.ds(r0, CROWS)])
        pltpu.sync_copy(cnt_v, cnt_out.at[wid])

    return k(features, labels2d, jnp.zeros((CP, D), jnp.float32))


def _tc_segment_sum(features, labels_row):
    steps = B_TC // K_TC
    off = B_SC // K_TC

    def body(lbl_ref, feat_ref, o_ref):
        step = pl.program_id(0)
        cls = lax.broadcasted_iota(jnp.int32, (CP, 1), 0)
        lbl = lbl_ref[...]
        oh = jnp.where(lbl == cls, 1.0, 0.0).astype(jnp.bfloat16)
        feat = feat_ref[...].astype(jnp.bfloat16)
        p = jax.lax.dot_general(
            oh, feat, (((1,), (0,)), ((), ())),
            preferred_element_type=jnp.float32)

        @pl.when(step == 0)
        def _():
            o_ref[...] = p

        @pl.when(step > 0)
        def _():
            o_ref[...] += p

    return pl.pallas_call(
        body,
        grid=(steps,),
        in_specs=[
            pl.BlockSpec((1, K_TC), lambda i: (0, off + i)),
            pl.BlockSpec((K_TC, D), lambda i: (off + i, 0)),
        ],
        out_specs=pl.BlockSpec((CP, D), lambda i: (0, 0)),
        out_shape=jax.ShapeDtypeStruct((CP, D), jnp.float32),
    )(labels_row, features)


def _combine(psum, tcsum, pcnt):
    HB = CP // 2

    def gbody(ps_ref, ts_ref, pc_ref, o_ref):
        s = ps_ref[0] + ps_ref[1] + ts_ref[...]
        c = jnp.sum(pc_ref[...], axis=0, keepdims=True)
        ct = jnp.transpose(c, (1, 0))
        o_ref[...] = jnp.where(ct > 0, s / jnp.maximum(ct, 1.0),
                               jnp.zeros_like(s))

    return pl.pallas_call(
        gbody,
        grid=(2,),
        in_specs=[
            pl.BlockSpec((NC, HB, D), lambda i: (0, i, 0)),
            pl.BlockSpec((HB, D), lambda i: (i, 0)),
            pl.BlockSpec((NW, HB), lambda i: (0, i)),
        ],
        out_specs=pl.BlockSpec((HB, D), lambda i: (i, 0)),
        out_shape=jax.ShapeDtypeStruct((C, D), jnp.float32),
    )(psum, tcsum, pcnt)


def kernel(features, labels):
    labels2d = labels.reshape(B // CHUNK, CHUNK)
    labels_row = labels.reshape(1, B)
    psum, pcnt = _sc_segment_sum(features, labels2d)
    tcsum = _tc_segment_sum(features, labels_row)
    return _combine(psum, tcsum, pcnt)


# speedup vs baseline: 1.0245x; 1.0245x over previous
import functools

import jax
import jax.numpy as jnp
from jax import lax
from jax.experimental import pallas as pl
from jax.experimental.pallas import tpu as pltpu
from jax.experimental.pallas import tpu_sc as plsc

B = 16384
D = 128
C = 1000
NC = 2
NS = 16
NW = NC * NS
CHUNK = 128
CP = 1024
CROWS = CP // NS

B_SC = 8192
B_TC = B - B_SC
ROWS_SC = B_SC // NW
NCHUNK_SC = ROWS_SC // CHUNK
LROWS = B // CHUNK // NW
K_TC = 2048


def _sc_segment_sum(features, labels2d):
    mesh = plsc.VectorSubcoreMesh(core_axis_name="c", subcore_axis_name="s")

    @functools.partial(
        pl.kernel,
        mesh=mesh,
        compiler_params=pltpu.CompilerParams(needs_layout_passes=False),
        out_type=(
            jax.ShapeDtypeStruct((NC, CP, D), jnp.float32),
            jax.ShapeDtypeStruct((NW, CP), jnp.float32),
        ),
        scratch_types=[
            pltpu.VMEM((ROWS_SC, D), jnp.float32),
            pltpu.VMEM((NCHUNK_SC, CHUNK), jnp.int32),
            pltpu.VMEM((LROWS, CHUNK), jnp.int32),
            pltpu.VMEM((CP,), jnp.float32),
            pltpu.VMEM((CROWS, D), jnp.float32),
            pltpu.VMEM_SHARED((CP, D), jnp.float32),
            pltpu.SemaphoreType.DMA,
            [pltpu.SemaphoreType.DMA] * NCHUNK_SC,
            pltpu.SemaphoreType.DMA,
        ],
    )
    def k(feat_hbm, lbl_hbm, sum_out, cnt_out,
          feat_v, lbls_v, lblc_v, cnt_v, zs_v, acc_s, sem_l, sems_f, sem_s):
        cid = lax.axis_index("c")
        sid = lax.axis_index("s")
        wid = cid * NS + sid
        base = wid * ROWS_SC

        lcop = pltpu.async_copy(
            lbl_hbm.at[pl.ds(wid * LROWS, LROWS)], lblc_v, sem_l)
        fcops = [
            pltpu.async_copy(
                feat_hbm.at[pl.ds(base + j * CHUNK, CHUNK)],
                feat_v.at[pl.ds(j * CHUNK, CHUNK)], sems_f[j])
            for j in range(NCHUNK_SC)
        ]
        scop = pltpu.async_copy(
            lbl_hbm.at[pl.ds(wid * NCHUNK_SC, NCHUNK_SC)], lbls_v, sem_l)

        zeros16 = jnp.zeros((16,), jnp.float32)
        ones16 = jnp.ones((16,), jnp.float32)

        def fill_zs(i, _):
            for j in range(D // 16):
                zs_v[i, pl.ds(j * 16, 16)] = zeros16
            return 0

        lax.fori_loop(0, CROWS, fill_zs, 0)

        def fill_zc(i, _):
            cnt_v[pl.ds(i * 16, 16)] = zeros16
            return 0

        lax.fori_loop(0, CP // 16, fill_zc, 0)

        pltpu.sync_copy(zs_v, acc_s.at[pl.ds(sid * CROWS, CROWS)])

        lcop.wait()

        def count_step(v, _):
            lbl16 = lblc_v[v // 8, pl.ds((v % 8) * 16, 16)]
            plsc.addupdate_scatter(cnt_v, [lbl16], ones16)
            return 0

        lax.fori_loop(0, LROWS * 8, count_step, 0)

        scop.wait()
        plsc.subcore_barrier()

        scats = []
        for j in range(NCHUNK_SC):
            fcops[j].wait()
            scats.append(pltpu.async_copy(
                feat_v.at[pl.ds(j * CHUNK, CHUNK)],
                acc_s.at[lbls_v.at[j]], sem_s, add=True))
        for s in scats:
            s.wait()

        plsc.subcore_barrier()

        r0 = sid * CROWS
        pltpu.sync_copy(acc_s.at[pl.ds(r0, CROWS)],
                        sum_out.at[cid, pl.ds(r0, CROWS)])
        pltpu.sync_copy(cnt_v, cnt_out.at[wid])

    return k(features, labels2d)


def _tc_segment_sum(features, labels_row):
    steps = B_TC // K_TC
    off = B_SC // K_TC

    def body(lbl_ref, feat_ref, o_ref):
        step = pl.program_id(0)
        cls = lax.broadcasted_iota(jnp.int32, (CP, 1), 0)
        lbl = lbl_ref[...]
        oh = jnp.where(lbl == cls, 1.0, 0.0).astype(jnp.bfloat16)
        feat = feat_ref[...].astype(jnp.bfloat16)
        p = jax.lax.dot_general(
            oh, feat, (((1,), (0,)), ((), ())),
            preferred_element_type=jnp.float32)

        @pl.when(step == 0)
        def _():
            o_ref[...] = p

        @pl.when(step > 0)
        def _():
            o_ref[...] += p

    return pl.pallas_call(
        body,
        grid=(steps,),
        in_specs=[
            pl.BlockSpec((1, K_TC), lambda i: (0, off + i)),
            pl.BlockSpec((K_TC, D), lambda i: (off + i, 0)),
        ],
        out_specs=pl.BlockSpec((CP, D), lambda i: (0, 0)),
        out_shape=jax.ShapeDtypeStruct((CP, D), jnp.float32),
    )(labels_row, features)


def _combine(psum, tcsum, pcnt):
    HB = CP // 2

    def gbody(ps_ref, ts_ref, pc_ref, o_ref):
        s = ps_ref[0] + ps_ref[1] + ts_ref[...]
        c = jnp.sum(pc_ref[...], axis=0, keepdims=True)
        ct = jnp.transpose(c, (1, 0))
        o_ref[...] = jnp.where(ct > 0, s / jnp.maximum(ct, 1.0),
                               jnp.zeros_like(s))

    return pl.pallas_call(
        gbody,
        grid=(2,),
        in_specs=[
            pl.BlockSpec((NC, HB, D), lambda i: (0, i, 0)),
            pl.BlockSpec((HB, D), lambda i: (i, 0)),
            pl.BlockSpec((NW, HB), lambda i: (0, i)),
        ],
        out_specs=pl.BlockSpec((HB, D), lambda i: (i, 0)),
        out_shape=jax.ShapeDtypeStruct((C, D), jnp.float32),
    )(psum, tcsum, pcnt)


def kernel(features, labels):
    labels2d = labels.reshape(B // CHUNK, CHUNK)
    labels_row = labels.reshape(1, B)
    psum, pcnt = _sc_segment_sum(features, labels2d)
    tcsum = _tc_segment_sum(features, labels_row)
    return _combine(psum, tcsum, pcnt)
